# baseline (speedup 1.0000x reference)
import jax
import jax.numpy as jnp
from jax.experimental import pallas as pl
from jax.experimental.pallas import tpu as pltpu


def _support_kernel(x_ref, w_ref, o_ref):
    xb = x_ref[...].astype(jnp.bfloat16)
    wb = w_ref[...].astype(jnp.bfloat16)
    o_ref[...] = jnp.dot(xb, wb, preferred_element_type=jnp.float32).astype(
        jnp.bfloat16)


def _spmm_bias_kernel(adj_ref, s_ref, b_ref, o_ref):
    a = adj_ref[...].astype(jnp.bfloat16)
    out = jnp.dot(a, s_ref[...], preferred_element_type=jnp.float32)
    o_ref[...] = out + b_ref[...]


def kernel(x, adj, weight, bias):
    N, S, F = x.shape
    M = N * S
    cols = S * F
    f_out = weight.shape[1]

    x2d = x.reshape(M, F)
    b_row = jnp.tile(bias, (S,)).reshape(1, cols).astype(jnp.float32)

    bm_a = 2048 if M % 2048 == 0 else M
    support = pl.pallas_call(
        _support_kernel,
        out_shape=jax.ShapeDtypeStruct((M, f_out), jnp.bfloat16),
        grid=(M // bm_a,),
        in_specs=[
            pl.BlockSpec((bm_a, F), lambda i: (i, 0)),
            pl.BlockSpec((F, f_out), lambda i: (0, 0)),
        ],
        out_specs=pl.BlockSpec((bm_a, f_out), lambda i: (i, 0)),
        compiler_params=pltpu.CompilerParams(
            dimension_semantics=("parallel",)),
    )(x2d, weight)

    s_flat = support.reshape(N, cols)

    bm = 256 if N % 256 == 0 else N
    out_flat = pl.pallas_call(
        _spmm_bias_kernel,
        out_shape=jax.ShapeDtypeStruct((N, cols), x.dtype),
        grid=(N // bm,),
        in_specs=[
            pl.BlockSpec((bm, N), lambda i: (i, 0)),
            pl.BlockSpec((N, cols), lambda i: (0, 0)),
            pl.BlockSpec((1, cols), lambda i: (0, 0)),
        ],
        out_specs=pl.BlockSpec((bm, cols), lambda i: (i, 0)),
        compiler_params=pltpu.CompilerParams(
            dimension_semantics=("parallel",)),
    )(adj, s_flat, b_row)

    return out_flat.reshape(N, S, F)


# speedup vs baseline: 1.0517x; 1.0517x over previous
import jax
import jax.numpy as jnp
from jax.experimental import pallas as pl
from jax.experimental.pallas import tpu as pltpu

_NC = 2


def _make_gcn_kernel(S, F):
    def _gcn_kernel(adj_ref, x_ref, w_ref, b_ref, o_ref, xb_ref):
        @pl.when(pl.program_id(1) == 0)
        def _cast_x():
            xb_ref[...] = x_ref[...].astype(jnp.bfloat16)

        a = adj_ref[...].astype(jnp.bfloat16)
        t = jnp.dot(a, xb_ref[...], preferred_element_type=jnp.float32)
        tb = t.astype(jnp.bfloat16)
        wb = w_ref[...].astype(jnp.bfloat16)
        b = b_ref[...]
        for s in range(S):
            sl = slice(s * F, (s + 1) * F)
            o_ref[:, sl] = jnp.dot(
                tb[:, sl], wb, preferred_element_type=jnp.float32) + b

    return _gcn_kernel


def kernel(x, adj, weight, bias):
    N, S, F = x.shape
    cols = S * F

    x_flat = x.reshape(N, cols)
    b_row = bias.reshape(1, F).astype(jnp.float32)

    bm = 256
    if N % (_NC * bm) != 0:
        bm = N // _NC if N % _NC == 0 else N
    nj = N // (_NC * bm) if bm != N else 1
    nc = _NC if bm != N else 1

    out_flat = pl.pallas_call(
        _make_gcn_kernel(S, F),
        out_shape=jax.ShapeDtypeStruct((N, cols), x.dtype),
        grid=(nc, nj),
        in_specs=[
            pl.BlockSpec((bm, N), lambda i, j, _nj=nj: (i * _nj + j, 0)),
            pl.BlockSpec((N, cols), lambda i, j: (0, 0)),
            pl.BlockSpec((F, F), lambda i, j: (0, 0)),
            pl.BlockSpec((1, F), lambda i, j: (0, 0)),
        ],
        out_specs=pl.BlockSpec(
            (bm, cols), lambda i, j, _nj=nj: (i * _nj + j, 0)),
        scratch_shapes=[pltpu.VMEM((N, cols), jnp.bfloat16)],
        compiler_params=pltpu.CompilerParams(
            dimension_semantics=("parallel", "arbitrary")),
    )(adj, x_flat, weight, b_row)

    return out_flat.reshape(N, S, F)


# speedup vs baseline: 2.4307x; 2.3112x over previous
import jax
import jax.numpy as jnp
from jax.experimental import pallas as pl
from jax.experimental.pallas import tpu as pltpu

_NC = 2


def _make_gcn_kernel(N, S, F):
    cols = S * F

    def _gcn_kernel(adj_ref, x_ref, w_ref, b_ref, o_ref, xb_ref):
        @pl.when(pl.program_id(1) == 0)
        def _prep():
            xb_ref[...] = x_ref[...].astype(jnp.bfloat16).reshape(N, cols)

        a = adj_ref[...].astype(jnp.bfloat16)
        t = jnp.dot(a, xb_ref[...], preferred_element_type=jnp.float32)
        tb = t.astype(jnp.bfloat16)
        wb = w_ref[...].astype(jnp.bfloat16)
        b = b_ref[...]
        for s in range(S):
            o_ref[:, s, :] = jnp.dot(
                tb[:, s * F:(s + 1) * F], wb,
                preferred_element_type=jnp.float32) + b

    return _gcn_kernel


def kernel(x, adj, weight, bias):
    N, S, F = x.shape
    cols = S * F

    x2d = x.reshape(N * S, F)
    b_row = bias.reshape(1, F).astype(jnp.float32)

    bm = 256
    if N % (_NC * bm) != 0:
        bm = N // _NC if N % _NC == 0 else N
    nj = N // (_NC * bm) if bm != N else 1
    nc = _NC if bm != N else 1

    return pl.pallas_call(
        _make_gcn_kernel(N, S, F),
        out_shape=jax.ShapeDtypeStruct((N, S, F), x.dtype),
        grid=(nc, nj),
        in_specs=[
            pl.BlockSpec((bm, N), lambda i, j, _nj=nj: (i * _nj + j, 0)),
            pl.BlockSpec((N * S, F), lambda i, j: (0, 0)),
            pl.BlockSpec((F, F), lambda i, j: (0, 0)),
            pl.BlockSpec((1, F), lambda i, j: (0, 0)),
        ],
        out_specs=pl.BlockSpec(
            (bm, S, F), lambda i, j, _nj=nj: (i * _nj + j, 0, 0)),
        scratch_shapes=[pltpu.VMEM((N, cols), jnp.bfloat16)],
        compiler_params=pltpu.CompilerParams(
            dimension_semantics=("parallel", "arbitrary")),
    )(adj, x2d, weight, b_row)


# speedup vs baseline: 2.5635x; 1.0546x over previous
import jax
import jax.numpy as jnp
from jax.experimental import pallas as pl
from jax.experimental.pallas import tpu as pltpu


def _make_gcn_kernel(N, S, F):
    cols = S * F

    def _gcn_kernel(adj_ref, x_ref, w_ref, b_ref, o_ref, xb_ref):
        @pl.when(pl.program_id(0) == 0)
        def _prep():
            xb_ref[...] = x_ref[...].astype(jnp.bfloat16).reshape(N, cols)

        a = adj_ref[...].astype(jnp.bfloat16)
        t = jnp.dot(a, xb_ref[...], preferred_element_type=jnp.float32)
        tb = t.astype(jnp.bfloat16)
        wb = w_ref[...].astype(jnp.bfloat16)
        b = b_ref[...]
        for s in range(S):
            o_ref[:, s, :] = jnp.dot(
                tb[:, s * F:(s + 1) * F], wb,
                preferred_element_type=jnp.float32) + b

    return _gcn_kernel


def kernel(x, adj, weight, bias):
    N, S, F = x.shape
    cols = S * F

    x2d = x.reshape(N * S, F)
    b_row = bias.reshape(1, F).astype(jnp.float32)

    bm = 256 if N % 256 == 0 else N

    return pl.pallas_call(
        _make_gcn_kernel(N, S, F),
        out_shape=jax.ShapeDtypeStruct((N, S, F), x.dtype),
        grid=(N // bm,),
        in_specs=[
            pl.BlockSpec((bm, N), lambda i: (i, 0)),
            pl.BlockSpec((N * S, F), lambda i: (0, 0)),
            pl.BlockSpec((F, F), lambda i: (0, 0)),
            pl.BlockSpec((1, F), lambda i: (0, 0)),
        ],
        out_specs=pl.BlockSpec((bm, S, F), lambda i: (i, 0, 0)),
        scratch_shapes=[pltpu.VMEM((N, cols), jnp.bfloat16)],
        compiler_params=pltpu.CompilerParams(
            dimension_semantics=("arbitrary",)),
    )(adj, x2d, weight, b_row)


# speedup vs baseline: 2.6198x; 1.0220x over previous
import jax
import jax.numpy as jnp
from jax.experimental import pallas as pl
from jax.experimental.pallas import tpu as pltpu


def _make_gcn_kernel(N, S, F):
    cols = S * F

    def _gcn_kernel(adj_ref, x_ref, w_ref, b_ref, o_ref, xb_ref):
        @pl.when(pl.program_id(0) == 0)
        def _prep():
            xb_ref[...] = x_ref[...].astype(jnp.bfloat16).reshape(N, cols)

        a = adj_ref[...].astype(jnp.bfloat16)
        t = jnp.dot(a, xb_ref[...], preferred_element_type=jnp.float32)
        tb = t.astype(jnp.bfloat16)
        wb = w_ref[...].astype(jnp.bfloat16)
        b = b_ref[...]
        for s in range(S):
            o_ref[:, s, :] = jnp.dot(
                tb[:, s * F:(s + 1) * F], wb,
                preferred_element_type=jnp.float32) + b

    return _gcn_kernel


def kernel(x, adj, weight, bias):
    N, S, F = x.shape
    cols = S * F

    x2d = x.reshape(N * S, F)
    b_row = bias.reshape(1, F).astype(jnp.float32)

    bm = 512 if N % 512 == 0 else N

    return pl.pallas_call(
        _make_gcn_kernel(N, S, F),
        out_shape=jax.ShapeDtypeStruct((N, S, F), x.dtype),
        grid=(N // bm,),
        in_specs=[
            pl.BlockSpec((bm, N), lambda i: (i, 0)),
            pl.BlockSpec((N * S, F), lambda i: (0, 0)),
            pl.BlockSpec((F, F), lambda i: (0, 0)),
            pl.BlockSpec((1, F), lambda i: (0, 0)),
        ],
        out_specs=pl.BlockSpec((bm, S, F), lambda i: (i, 0, 0)),
        scratch_shapes=[pltpu.VMEM((N, cols), jnp.bfloat16)],
        compiler_params=pltpu.CompilerParams(
            dimension_semantics=("arbitrary",)),
    )(adj, x2d, weight, b_row)
